# Initial kernel scaffold; baseline (speedup 1.0000x reference)
import jax
import jax.numpy as jnp
from jax.experimental import pallas as pl


def kernel(X, shuffled_idx, rand_idx):
    raise NotImplementedError("write your pallas kernel here")


# speedup vs baseline: 5.1028x; 5.1028x over previous
import functools

import jax
import jax.numpy as jnp
from jax import lax
from jax.experimental import pallas as pl
from jax.experimental.pallas import tpu as pltpu
from jax.experimental.pallas import tpu_sc as plsc

_B, _C, _E, _P = 64, 16, 256, 128
_NBLK = _B * _C
_NW = 32
_BLK_PER_W = _NBLK // _NW
_CHUNK = 64
_NCHUNK = _E // _CHUNK
_LANE = 16


def _sc_shuffle(x3, shuffled_idx, rand_idx):
    mesh = plsc.VectorSubcoreMesh(
        core_axis_name="c", subcore_axis_name="s", num_cores=2, num_subcores=16)

    @functools.partial(
        pl.kernel,
        out_type=jax.ShapeDtypeStruct((_NBLK, _E, _P), jnp.float32),
        mesh=mesh,
        scratch_types=[
            pltpu.VMEM((1,), jnp.int32),
            pltpu.VMEM((1, _C, _P), jnp.int32),
            pltpu.VMEM((_CHUNK, _P), jnp.float32),
            pltpu.VMEM((_CHUNK, _P), jnp.float32),
            pltpu.SemaphoreType.DMA,
        ],
        compiler_params=pltpu.CompilerParams(needs_layout_passes=False),
    )
    def k(x_hbm, sidx_hbm, ridx_hbm, out_hbm, ridx_v, idx_v, in_v, out_v, sem):
        wid = lax.axis_index("s") * 2 + lax.axis_index("c")
        pltpu.sync_copy(ridx_hbm, ridx_v)
        pltpu.async_copy(sidx_hbm.at[ridx_v], idx_v, sem).wait()

        def body_g(g, carry):
            blk = wid * _BLK_PER_W + g
            ch = lax.rem(blk, _C)
            idxs = [idx_v[0, ch, pl.ds(_LANE * j, _LANE)] for j in range(_P // _LANE)]

            def body_t(t, carry_t):
                pltpu.sync_copy(x_hbm.at[blk, pl.ds(t * _CHUNK, _CHUNK)], in_v)

                def body_r(rw, carry_r):
                    rowv = jnp.full((_LANE,), rw, jnp.int32)
                    for j in range(_P // _LANE):
                        out_v[rw, pl.ds(_LANE * j, _LANE)] = plsc.load_gather(
                            in_v, [rowv, idxs[j]])
                    return carry_r

                lax.fori_loop(0, _CHUNK, body_r, 0, unroll=False)
                pltpu.sync_copy(out_v, out_hbm.at[blk, pl.ds(t * _CHUNK, _CHUNK)])
                return carry_t

            lax.fori_loop(0, _NCHUNK, body_t, 0, unroll=False)
            return carry

        lax.fori_loop(0, _BLK_PER_W, body_g, 0, unroll=False)

    return k(x3, shuffled_idx, rand_idx)


def kernel(X, shuffled_idx, rand_idx):
    x3 = X.reshape(_NBLK, _E, _P)
    out = _sc_shuffle(x3, shuffled_idx, rand_idx.astype(jnp.int32))
    return out.reshape(_B, _C, _E, _P)
